# Initial kernel scaffold; baseline (speedup 1.0000x reference)
import jax
import jax.numpy as jnp
from jax.experimental import pallas as pl


def kernel(triples, neighbor_indices, segment_ids, relation_table):
    raise NotImplementedError("write your pallas kernel here")


# speedup vs baseline: 36.6967x; 36.6967x over previous
import functools

import jax
import jax.numpy as jnp
from jax import lax
from jax.experimental import pallas as pl
from jax.experimental.pallas import tpu as pltpu
from jax.experimental.pallas import tpu_sc as plsc

_N, _M, _T, _NUM_REL, _D = 16, 256, 131072, 2000, 256
_P = _N * _M
_RPAD = 2048
_NW = 32
_CHUNK = _T // _NW
_NBLK = _CHUNK // 16
_GCH = 128
_SQD = 16.0


def _gram_body(a_ref, b_ref, o_ref):
    o_ref[...] = lax.dot_general(
        a_ref[...], b_ref[...], (((1,), (1,)), ((), ())),
        preferred_element_type=jnp.float32) * (1.0 / _SQD)


def _gram(r_pad):
    blk = 256
    return pl.pallas_call(
        _gram_body,
        grid=(_RPAD // blk, _RPAD // blk),
        in_specs=[
            pl.BlockSpec((blk, _D), lambda i, j: (i, 0)),
            pl.BlockSpec((blk, _D), lambda i, j: (j, 0)),
        ],
        out_specs=pl.BlockSpec((blk, blk), lambda i, j: (i, j)),
        out_shape=jax.ShapeDtypeStruct((_RPAD, _RPAD), jnp.float32),
    )(r_pad, r_pad)


def _combine_body(p_ref, o_ref):
    den = jnp.sum(p_ref[:, 0, :, :], axis=0)
    num = jnp.sum(p_ref[:, 1, :, :], axis=0)
    o_ref[...] = _SQD * num / (den + 1e-9)


def _combine(partials):
    return pl.pallas_call(
        _combine_body,
        out_shape=jax.ShapeDtypeStruct((_N, _M), jnp.float32),
    )(partials)


def _sc_body(g_ref, nbr_ref, seg_ref, cand_ref, out_ref,
             segbuf, nbr_v, idx_v, logit_v, cand_v, a_den, a_num, sem):
    wid = lax.axis_index("s") * 2 + lax.axis_index("c")
    base = wid * _CHUNK

    pltpu.sync_copy(seg_ref.at[pl.ds(base, _CHUNK)], segbuf.at[pl.ds(16, _CHUNK)])
    pltpu.sync_copy(nbr_ref.at[pl.ds(base, _CHUNK)], nbr_v)
    pltpu.sync_copy(cand_ref, cand_v)
    segbuf[pl.ds(0, 16)] = jnp.full((16,), -1, jnp.int32)
    segbuf[pl.ds(16 + _CHUNK, 16)] = jnp.full((16,), -2, jnp.int32)

    zeros = jnp.zeros((16,), jnp.float32)

    def init_body(i, c):
        a_den[pl.ds(i * 16, 16)] = zeros
        a_num[pl.ds(i * 16, 16)] = zeros
        s = segbuf[pl.ds(16 + i * 16, 16)]
        nb = nbr_v[pl.ds(i * 16, 16)]
        cd = plsc.load_gather(cand_v, [s])
        idx_v[pl.ds(i * 16, 16)] = nb * _RPAD + cd
        return c

    lax.fori_loop(0, _NBLK, init_body, 0)

    copies = []
    for j in range(_CHUNK // _GCH):
        copies.append(pltpu.async_copy(
            g_ref.at[idx_v.at[pl.ds(j * _GCH, _GCH)]],
            logit_v.at[pl.ds(j * _GCH, _GCH)], sem))
    for cp in copies:
        cp.wait()

    it = lax.iota(jnp.int32, 16)

    def body(i, carry):
        ce_c, cn_c = carry
        s = segbuf[pl.ds(16 + i * 16, 16)]
        sp = plsc.load_gather(segbuf, [it + (15 + i * 16)])
        sn = plsc.load_gather(segbuf, [it + (17 + i * 16)])
        l = logit_v[pl.ds(i * 16, 16)]
        e = jnp.exp(l)
        en = e * l
        ce = plsc.cumsum(e) + ce_c
        cn = plsc.cumsum(en) + cn_c
        start = s != sp
        end = s != sn
        vd = jnp.where(end, ce, 0.0) + jnp.where(start, e - ce, 0.0)
        vn = jnp.where(end, cn, 0.0) + jnp.where(start, en - cn, 0.0)
        m = start | end
        plsc.addupdate_scatter(a_den, [s], vd, mask=m)
        plsc.addupdate_scatter(a_num, [s], vn, mask=m)
        return (ce_c + jnp.sum(e), cn_c + jnp.sum(en))

    lax.fori_loop(0, _NBLK, body,
                  (jnp.zeros((), jnp.float32), jnp.zeros((), jnp.float32)))

    pltpu.sync_copy(a_den, out_ref.at[wid, 0])
    pltpu.sync_copy(a_num, out_ref.at[wid, 1])


def _sc_partials(g_flat, nbr, seg, cand_idx):
    mesh = plsc.VectorSubcoreMesh(core_axis_name="c", subcore_axis_name="s")
    f = functools.partial(
        pl.kernel, mesh=mesh,
        compiler_params=pltpu.CompilerParams(needs_layout_passes=False),
        out_type=jax.ShapeDtypeStruct((_NW, 2, _P), jnp.float32),
        scratch_types=[
            pltpu.VMEM((_CHUNK + 32,), jnp.int32),
            pltpu.VMEM((_CHUNK,), jnp.int32),
            pltpu.VMEM((_CHUNK,), jnp.int32),
            pltpu.VMEM((_CHUNK,), jnp.float32),
            pltpu.VMEM((_P,), jnp.int32),
            pltpu.VMEM((_P,), jnp.float32),
            pltpu.VMEM((_P,), jnp.float32),
            pltpu.SemaphoreType.DMA,
        ],
    )(_sc_body)
    return f(g_flat, nbr, seg, cand_idx)


@jax.jit
def _impl(triples, neighbor_indices, segment_ids, relation_table):
    cand_idx = triples[:, :, 2].reshape(-1).astype(jnp.int32)
    nbr = neighbor_indices.astype(jnp.int32)
    seg = segment_ids.astype(jnp.int32)
    r_pad = jnp.zeros((_RPAD, _D), jnp.float32).at[:_NUM_REL, :].set(relation_table)
    g_flat = _gram(r_pad).reshape(-1)
    partials = _sc_partials(g_flat, nbr, seg, cand_idx)
    return _combine(partials.reshape(_NW, 2, _N, _M))


def kernel(triples, neighbor_indices, segment_ids, relation_table):
    return _impl(triples, neighbor_indices, segment_ids, relation_table)


# speedup vs baseline: 57.9899x; 1.5802x over previous
import functools

import jax
import jax.numpy as jnp
from jax import lax
from jax.experimental import pallas as pl
from jax.experimental.pallas import tpu as pltpu
from jax.experimental.pallas import tpu_sc as plsc

_N, _M, _T, _NUM_REL, _D = 16, 256, 131072, 2000, 256
_P = _N * _M
_RPAD = 2048
_NW = 32
_CHUNK = _T // _NW
_NBLK = _CHUNK // 16
_GCH = 128
_SQD = 16.0


def _gram_body(a_ref, b_ref, o_ref):
    o_ref[...] = lax.dot_general(
        a_ref[...], b_ref[...], (((1,), (1,)), ((), ())),
        preferred_element_type=jnp.float32) * (1.0 / _SQD)


def _gram(r):
    blk = 256
    return pl.pallas_call(
        _gram_body,
        grid=(_RPAD // blk,),
        in_specs=[
            pl.BlockSpec((blk, _D), lambda i: (i, 0)),
            pl.BlockSpec((_RPAD, _D), lambda i: (0, 0)),
        ],
        out_specs=pl.BlockSpec((blk, _RPAD), lambda i: (i, 0)),
        out_shape=jax.ShapeDtypeStruct((_RPAD, _RPAD), jnp.float32),
    )(r, r)


def _combine_body(p_ref, o_ref):
    den = jnp.sum(p_ref[:, 0, :, :], axis=0)
    num = jnp.sum(p_ref[:, 1, :, :], axis=0)
    o_ref[...] = _SQD * num / (den + 1e-9)


def _combine(partials):
    return pl.pallas_call(
        _combine_body,
        out_shape=jax.ShapeDtypeStruct((_N, _M), jnp.float32),
    )(partials)


def _sc_body(g_ref, nbr_ref, seg_ref, cand_ref, out_ref,
             segbuf, nbr_v, idx_v, logit_v, cand_v, a_den, a_num, sem):
    wid = lax.axis_index("s") * 2 + lax.axis_index("c")
    base = wid * _CHUNK

    pltpu.sync_copy(seg_ref.at[pl.ds(base, _CHUNK)], segbuf.at[pl.ds(16, _CHUNK)])
    pltpu.sync_copy(nbr_ref.at[pl.ds(base, _CHUNK)], nbr_v)
    pltpu.sync_copy(cand_ref, cand_v)
    segbuf[pl.ds(0, 16)] = jnp.full((16,), -1, jnp.int32)
    segbuf[pl.ds(16 + _CHUNK, 16)] = jnp.full((16,), -2, jnp.int32)

    zeros = jnp.zeros((16,), jnp.float32)

    def init_body(i, c):
        a_den[pl.ds(i * 16, 16)] = zeros
        a_num[pl.ds(i * 16, 16)] = zeros
        s = segbuf[pl.ds(16 + i * 16, 16)]
        nb = nbr_v[pl.ds(i * 16, 16)]
        cd = plsc.load_gather(cand_v, [s])
        idx_v[pl.ds(i * 16, 16)] = nb * _RPAD + cd
        return c

    lax.fori_loop(0, _NBLK, init_body, 0)

    copies = []
    for j in range(_CHUNK // _GCH):
        copies.append(pltpu.async_copy(
            g_ref.at[idx_v.at[pl.ds(j * _GCH, _GCH)]],
            logit_v.at[pl.ds(j * _GCH, _GCH)], sem))
    for cp in copies:
        cp.wait()

    it = lax.iota(jnp.int32, 16)

    def body(i, carry):
        ce_c, cn_c = carry
        s = segbuf[pl.ds(16 + i * 16, 16)]
        sp = plsc.load_gather(segbuf, [it + (15 + i * 16)])
        sn = plsc.load_gather(segbuf, [it + (17 + i * 16)])
        l = logit_v[pl.ds(i * 16, 16)]
        e = jnp.exp(l)
        en = e * l
        ce = plsc.cumsum(e) + ce_c
        cn = plsc.cumsum(en) + cn_c
        start = s != sp
        end = s != sn
        vd = jnp.where(end, ce, 0.0) + jnp.where(start, e - ce, 0.0)
        vn = jnp.where(end, cn, 0.0) + jnp.where(start, en - cn, 0.0)
        m = start | end
        plsc.addupdate_scatter(a_den, [s], vd, mask=m)
        plsc.addupdate_scatter(a_num, [s], vn, mask=m)
        return (ce_c + jnp.sum(e), cn_c + jnp.sum(en))

    lax.fori_loop(0, _NBLK, body,
                  (jnp.zeros((), jnp.float32), jnp.zeros((), jnp.float32)))

    pltpu.sync_copy(a_den, out_ref.at[wid, 0])
    pltpu.sync_copy(a_num, out_ref.at[wid, 1])


def _sc_partials(g_flat, nbr, seg, cand_idx):
    mesh = plsc.VectorSubcoreMesh(core_axis_name="c", subcore_axis_name="s")
    f = functools.partial(
        pl.kernel, mesh=mesh,
        compiler_params=pltpu.CompilerParams(needs_layout_passes=False),
        out_type=jax.ShapeDtypeStruct((_NW, 2, _P), jnp.float32),
        scratch_types=[
            pltpu.VMEM((_CHUNK + 32,), jnp.int32),
            pltpu.VMEM((_CHUNK,), jnp.int32),
            pltpu.VMEM((_CHUNK,), jnp.int32),
            pltpu.VMEM((_CHUNK,), jnp.float32),
            pltpu.VMEM((_P,), jnp.int32),
            pltpu.VMEM((_P,), jnp.float32),
            pltpu.VMEM((_P,), jnp.float32),
            pltpu.SemaphoreType.DMA,
        ],
    )(_sc_body)
    return f(g_flat, nbr, seg, cand_idx)


@jax.jit
def _impl(triples, neighbor_indices, segment_ids, relation_table):
    cand_idx = triples[:, :, 2].reshape(-1).astype(jnp.int32)
    nbr = neighbor_indices.astype(jnp.int32)
    seg = segment_ids.astype(jnp.int32)
    g_flat = _gram(relation_table).reshape(-1)
    partials = _sc_partials(g_flat, nbr, seg, cand_idx)
    return _combine(partials.reshape(_NW, 2, _N, _M))


def kernel(triples, neighbor_indices, segment_ids, relation_table):
    return _impl(triples, neighbor_indices, segment_ids, relation_table)


# speedup vs baseline: 61.7768x; 1.0653x over previous
import functools

import jax
import jax.numpy as jnp
from jax import lax
from jax.experimental import pallas as pl
from jax.experimental.pallas import tpu as pltpu
from jax.experimental.pallas import tpu_sc as plsc

_N, _M, _T, _NUM_REL, _D = 16, 256, 131072, 2000, 256
_P = _N * _M
_RPAD = 2048
_NW = 32
_CHUNK = _T // _NW
_NBLK = _CHUNK // 16
_GCH = 128
_SQD = 16.0


def _gram_body(a_ref, b_ref, o_ref):
    o_ref[...] = lax.dot_general(
        a_ref[...], b_ref[...], (((1,), (1,)), ((), ())),
        preferred_element_type=jnp.float32) * (1.0 / _SQD)


def _gram(r):
    blk = 256
    return pl.pallas_call(
        _gram_body,
        grid=(_RPAD // blk,),
        in_specs=[
            pl.BlockSpec((blk, _D), lambda i: (i, 0)),
            pl.BlockSpec((_RPAD, _D), lambda i: (0, 0)),
        ],
        out_specs=pl.BlockSpec((blk, _RPAD), lambda i: (i, 0)),
        out_shape=jax.ShapeDtypeStruct((_RPAD, _RPAD), jnp.float32),
    )(r, r)


def _combine_body(p_ref, o_ref):
    den = jnp.sum(p_ref[:, 0, :, :], axis=0)
    num = jnp.sum(p_ref[:, 1, :, :], axis=0)
    o_ref[...] = _SQD * num / (den + 1e-9)


def _combine(partials):
    return pl.pallas_call(
        _combine_body,
        out_shape=jax.ShapeDtypeStruct((_N, _M), jnp.float32),
    )(partials)


def _last_lane(x):
    return lax.squeeze(lax.slice(x, (15,), (16,)), dimensions=(0,))


def _sc_body(g_ref, nbr_ref, seg_ref, cand_ref, out_ref,
             segbuf, nbr_v, idx_v, logit_v, cand_v, a_den, a_num,
             ce_buf, ex_buf, cn_buf, exn_buf, offs_e, offs_n, sem):
    wid = lax.axis_index("s") * 2 + lax.axis_index("c")
    base = wid * _CHUNK

    in_cp = [
        pltpu.async_copy(seg_ref.at[pl.ds(base, _CHUNK)],
                         segbuf.at[pl.ds(16, _CHUNK)], sem),
        pltpu.async_copy(nbr_ref.at[pl.ds(base, _CHUNK)], nbr_v, sem),
        pltpu.async_copy(cand_ref, cand_v, sem),
    ]
    for cp in in_cp:
        cp.wait()
    segbuf[pl.ds(0, 16)] = jnp.full((16,), -1, jnp.int32)
    segbuf[pl.ds(16 + _CHUNK, 16)] = jnp.full((16,), -2, jnp.int32)

    @plsc.parallel_loop(0, _NBLK, unroll=4)
    def _idx_loop(i):
        s = segbuf[pl.ds(16 + i * 16, 16)]
        nb = nbr_v[pl.ds(i * 16, 16)]
        cd = plsc.load_gather(cand_v, [s])
        idx_v[pl.ds(i * 16, 16)] = nb * _RPAD + cd

    copies = []
    for j in range(_CHUNK // _GCH):
        copies.append(pltpu.async_copy(
            g_ref.at[idx_v.at[pl.ds(j * _GCH, _GCH)]],
            logit_v.at[pl.ds(j * _GCH, _GCH)], sem))

    zeros = jnp.zeros((16,), jnp.float32)

    @plsc.parallel_loop(0, _NBLK, unroll=8)
    def _zero_loop(i):
        a_den[pl.ds(i * 16, 16)] = zeros
        a_num[pl.ds(i * 16, 16)] = zeros

    for cp in copies:
        cp.wait()

    @plsc.parallel_loop(0, _NBLK, unroll=4)
    def _cum_loop(i):
        l = logit_v[pl.ds(i * 16, 16)]
        e = jnp.exp(l)
        en = e * l
        ce = plsc.cumsum(e)
        cn = plsc.cumsum(en)
        ce_buf[pl.ds(i * 16, 16)] = ce
        ex_buf[pl.ds(i * 16, 16)] = ce - e
        cn_buf[pl.ds(i * 16, 16)] = cn
        exn_buf[pl.ds(i * 16, 16)] = cn - en

    it = lax.iota(jnp.int32, 16)

    def bstep(k, carry):
        c_e, c_n = carry
        ids = it * 16 + (15 + 256 * k)
        ve = plsc.load_gather(ce_buf, [ids])
        vn = plsc.load_gather(cn_buf, [ids])
        ie = plsc.cumsum(ve) + c_e
        inn = plsc.cumsum(vn) + c_n
        offs_e[pl.ds(k * 16, 16)] = ie - ve
        offs_n[pl.ds(k * 16, 16)] = inn - vn
        return (_last_lane(ie), _last_lane(inn))

    lax.fori_loop(0, _NBLK // 16, bstep,
                  (jnp.zeros((), jnp.float32), jnp.zeros((), jnp.float32)))

    @plsc.parallel_loop(0, _NBLK, unroll=4)
    def _scatter_loop(i):
        s = segbuf[pl.ds(16 + i * 16, 16)]
        sp = plsc.load_gather(segbuf, [it + (15 + i * 16)])
        sn = plsc.load_gather(segbuf, [it + (17 + i * 16)])
        bi = jnp.full((16,), 0, jnp.int32) + i
        o_e = plsc.load_gather(offs_e, [bi])
        o_n = plsc.load_gather(offs_n, [bi])
        ce = ce_buf[pl.ds(i * 16, 16)] + o_e
        exl = ex_buf[pl.ds(i * 16, 16)] + o_e
        cn = cn_buf[pl.ds(i * 16, 16)] + o_n
        exn = exn_buf[pl.ds(i * 16, 16)] + o_n
        start = s != sp
        end = s != sn
        vd = jnp.where(end, ce, 0.0) - jnp.where(start, exl, 0.0)
        vn = jnp.where(end, cn, 0.0) - jnp.where(start, exn, 0.0)
        m = start | end
        plsc.addupdate_scatter(a_den, [s], vd, mask=m)
        plsc.addupdate_scatter(a_num, [s], vn, mask=m)

    pltpu.sync_copy(a_den, out_ref.at[wid, 0])
    pltpu.sync_copy(a_num, out_ref.at[wid, 1])


def _sc_partials(g_flat, nbr, seg, cand_idx):
    mesh = plsc.VectorSubcoreMesh(core_axis_name="c", subcore_axis_name="s")
    f = functools.partial(
        pl.kernel, mesh=mesh,
        compiler_params=pltpu.CompilerParams(needs_layout_passes=False),
        out_type=jax.ShapeDtypeStruct((_NW, 2, _P), jnp.float32),
        scratch_types=[
            pltpu.VMEM((_CHUNK + 32,), jnp.int32),
            pltpu.VMEM((_CHUNK,), jnp.int32),
            pltpu.VMEM((_CHUNK,), jnp.int32),
            pltpu.VMEM((_CHUNK,), jnp.float32),
            pltpu.VMEM((_P,), jnp.int32),
            pltpu.VMEM((_P,), jnp.float32),
            pltpu.VMEM((_P,), jnp.float32),
            pltpu.VMEM((_CHUNK,), jnp.float32),
            pltpu.VMEM((_CHUNK,), jnp.float32),
            pltpu.VMEM((_CHUNK,), jnp.float32),
            pltpu.VMEM((_CHUNK,), jnp.float32),
            pltpu.VMEM((_NBLK,), jnp.float32),
            pltpu.VMEM((_NBLK,), jnp.float32),
            pltpu.SemaphoreType.DMA,
        ],
    )(_sc_body)
    return f(g_flat, nbr, seg, cand_idx)


@jax.jit
def _impl(triples, neighbor_indices, segment_ids, relation_table):
    cand_idx = triples[:, :, 2].reshape(-1).astype(jnp.int32)
    nbr = neighbor_indices.astype(jnp.int32)
    seg = segment_ids.astype(jnp.int32)
    g_flat = _gram(relation_table).reshape(-1)
    partials = _sc_partials(g_flat, nbr, seg, cand_idx)
    return _combine(partials.reshape(_NW, 2, _N, _M))


def kernel(triples, neighbor_indices, segment_ids, relation_table):
    return _impl(triples, neighbor_indices, segment_ids, relation_table)


# speedup vs baseline: 86.7938x; 1.4050x over previous
import functools

import jax
import jax.numpy as jnp
from jax import lax
from jax.experimental import pallas as pl
from jax.experimental.pallas import tpu as pltpu
from jax.experimental.pallas import tpu_sc as plsc

_N, _M, _T, _NUM_REL, _D = 16, 256, 131072, 2000, 256
_P = _N * _M
_RPAD = 2048
_NW = 32
_CHUNK = _T // _NW
_NBLK = _CHUNK // 16
_GCH = 128
_SQD = 16.0


def _gram_body(a_ref, b_ref, o_ref):
    res = lax.dot_general(
        a_ref[...], b_ref[...], (((1,), (1,)), ((), ())),
        preferred_element_type=jnp.float32) * (1.0 / _SQD)
    for ct in range(_RPAD // 128):
        o_ref[:, ct] = res[:, ct * 128:(ct + 1) * 128].reshape(-1, 8, 128)


def _gram(r):
    blk = 256
    return pl.pallas_call(
        _gram_body,
        grid=(_RPAD // blk,),
        in_specs=[
            pl.BlockSpec((blk, _D), lambda i: (i, 0)),
            pl.BlockSpec((_RPAD, _D), lambda i: (0, 0)),
        ],
        out_specs=pl.BlockSpec((blk // 8, _RPAD // 128, 8, 128),
                               lambda i: (i, 0, 0, 0)),
        out_shape=jax.ShapeDtypeStruct(
            (_RPAD // 8, _RPAD // 128, 8, 128), jnp.float32),
    )(r, r)


def _combine_body(p_ref, o_ref):
    p = p_ref[...].reshape(_NW, 2, _P // 128, 128)
    den = jnp.sum(p[:, 0], axis=0)
    num = jnp.sum(p[:, 1], axis=0)
    o_ref[...] = _SQD * num / (den + 1e-9)


def _combine(partials):
    return pl.pallas_call(
        _combine_body,
        in_specs=[pl.BlockSpec((_NW * 2 * (_P // 128), 128),
                               lambda: (0, 0))],
        out_specs=pl.BlockSpec((_P // 128, 128), lambda: (0, 0)),
        out_shape=jax.ShapeDtypeStruct((_P // 128, 128), jnp.float32),
    )(partials.reshape(_NW * 2 * (_P // 128), 128))


def _last_lane(x):
    return lax.squeeze(lax.slice(x, (15,), (16,)), dimensions=(0,))


def _sc_body(g_ref, nbr_ref, seg_ref, cand_ref, out_ref,
             segbuf, nbr_v, idx_v, logit_v, cand_v, a_den, a_num,
             ce_buf, ex_buf, cn_buf, exn_buf, offs_e, offs_n, sem):
    wid = lax.axis_index("s") * 2 + lax.axis_index("c")
    base = wid * _CHUNK

    in_cp = [
        pltpu.async_copy(seg_ref.at[pl.ds(base, _CHUNK)],
                         segbuf.at[pl.ds(16, _CHUNK)], sem),
        pltpu.async_copy(nbr_ref.at[pl.ds(base, _CHUNK)], nbr_v, sem),
        pltpu.async_copy(cand_ref, cand_v, sem),
    ]
    for cp in in_cp:
        cp.wait()
    segbuf[pl.ds(0, 16)] = jnp.full((16,), -1, jnp.int32)
    segbuf[pl.ds(16 + _CHUNK, 16)] = jnp.full((16,), -2, jnp.int32)

    @plsc.parallel_loop(0, _NBLK, unroll=4)
    def _idx_loop(i):
        s = segbuf[pl.ds(16 + i * 16, 16)]
        nb = nbr_v[pl.ds(i * 16, 16)]
        cd = plsc.load_gather(cand_v, [s])
        idx_v[pl.ds(i * 16, 16)] = (
            ((nb >> 3) << 14) + ((cd >> 7) << 10) + ((nb & 7) << 7)
            + (cd & 127))

    copies = []
    for j in range(_CHUNK // _GCH):
        copies.append(pltpu.async_copy(
            g_ref.at[idx_v.at[pl.ds(j * _GCH, _GCH)]],
            logit_v.at[pl.ds(j * _GCH, _GCH)], sem))

    zeros = jnp.zeros((16,), jnp.float32)

    @plsc.parallel_loop(0, _NBLK, unroll=8)
    def _zero_loop(i):
        a_den[pl.ds(i * 16, 16)] = zeros
        a_num[pl.ds(i * 16, 16)] = zeros

    for cp in copies:
        cp.wait()

    @plsc.parallel_loop(0, _NBLK, unroll=4)
    def _cum_loop(i):
        l = logit_v[pl.ds(i * 16, 16)]
        e = jnp.exp(l)
        en = e * l
        ce = plsc.cumsum(e)
        cn = plsc.cumsum(en)
        ce_buf[pl.ds(i * 16, 16)] = ce
        ex_buf[pl.ds(i * 16, 16)] = ce - e
        cn_buf[pl.ds(i * 16, 16)] = cn
        exn_buf[pl.ds(i * 16, 16)] = cn - en

    it = lax.iota(jnp.int32, 16)

    def bstep(k, carry):
        c_e, c_n = carry
        ids = it * 16 + (15 + 256 * k)
        ve = plsc.load_gather(ce_buf, [ids])
        vn = plsc.load_gather(cn_buf, [ids])
        ie = plsc.cumsum(ve) + c_e
        inn = plsc.cumsum(vn) + c_n
        offs_e[pl.ds(k * 16, 16)] = ie - ve
        offs_n[pl.ds(k * 16, 16)] = inn - vn
        return (_last_lane(ie), _last_lane(inn))

    lax.fori_loop(0, _NBLK // 16, bstep,
                  (jnp.zeros((), jnp.float32), jnp.zeros((), jnp.float32)))

    @plsc.parallel_loop(0, _NBLK, unroll=4)
    def _scatter_loop(i):
        s = segbuf[pl.ds(16 + i * 16, 16)]
        sp = plsc.load_gather(segbuf, [it + (15 + i * 16)])
        sn = plsc.load_gather(segbuf, [it + (17 + i * 16)])
        bi = jnp.full((16,), 0, jnp.int32) + i
        o_e = plsc.load_gather(offs_e, [bi])
        o_n = plsc.load_gather(offs_n, [bi])
        ce = ce_buf[pl.ds(i * 16, 16)] + o_e
        exl = ex_buf[pl.ds(i * 16, 16)] + o_e
        cn = cn_buf[pl.ds(i * 16, 16)] + o_n
        exn = exn_buf[pl.ds(i * 16, 16)] + o_n
        start = s != sp
        end = s != sn
        vd = jnp.where(end, ce, 0.0) - jnp.where(start, exl, 0.0)
        vn = jnp.where(end, cn, 0.0) - jnp.where(start, exn, 0.0)
        m = start | end
        plsc.addupdate_scatter(a_den, [s], vd, mask=m)
        plsc.addupdate_scatter(a_num, [s], vn, mask=m)

    pltpu.sync_copy(a_den, out_ref.at[pl.ds(wid * 2 * _P, _P)])
    pltpu.sync_copy(a_num, out_ref.at[pl.ds(wid * 2 * _P + _P, _P)])


def _sc_partials(g_flat, nbr, seg, cand_idx):
    mesh = plsc.VectorSubcoreMesh(core_axis_name="c", subcore_axis_name="s")
    f = functools.partial(
        pl.kernel, mesh=mesh,
        compiler_params=pltpu.CompilerParams(needs_layout_passes=False),
        out_type=jax.ShapeDtypeStruct((_NW * 2 * _P,), jnp.float32),
        scratch_types=[
            pltpu.VMEM((_CHUNK + 32,), jnp.int32),
            pltpu.VMEM((_CHUNK,), jnp.int32),
            pltpu.VMEM((_CHUNK,), jnp.int32),
            pltpu.VMEM((_CHUNK,), jnp.float32),
            pltpu.VMEM((_P,), jnp.int32),
            pltpu.VMEM((_P,), jnp.float32),
            pltpu.VMEM((_P,), jnp.float32),
            pltpu.VMEM((_CHUNK,), jnp.float32),
            pltpu.VMEM((_CHUNK,), jnp.float32),
            pltpu.VMEM((_CHUNK,), jnp.float32),
            pltpu.VMEM((_CHUNK,), jnp.float32),
            pltpu.VMEM((_NBLK,), jnp.float32),
            pltpu.VMEM((_NBLK,), jnp.float32),
            pltpu.SemaphoreType.DMA,
        ],
    )(_sc_body)
    return f(g_flat, nbr, seg, cand_idx)


@jax.jit
def _impl(triples, neighbor_indices, segment_ids, relation_table):
    cand_idx = triples[:, :, 2].reshape(-1).astype(jnp.int32)
    nbr = neighbor_indices.astype(jnp.int32)
    seg = segment_ids.astype(jnp.int32)
    g_flat = _gram(relation_table).reshape(-1)
    partials = _sc_partials(g_flat, nbr, seg, cand_idx)
    return _combine(partials).reshape(_N, _M)


def kernel(triples, neighbor_indices, segment_ids, relation_table):
    return _impl(triples, neighbor_indices, segment_ids, relation_table)
